# baseline (speedup 1.0000x reference)
import functools

import jax
import jax.numpy as jnp
from jax import lax
from jax.experimental import pallas as pl
from jax.experimental.pallas import tpu as pltpu
from jax.experimental.pallas import tpu_sc as plsc

T = 16
K = 8
ROW = 512 * 512 * 8
NC, NS = 2, 16
NW = NC * NS
CH = 256
CW = ROW // CH
CPW = CH // NW

_mesh = plsc.VectorSubcoreMesh(core_axis_name="c", subcore_axis_name="s")


@functools.partial(
    pl.kernel,
    out_type=jax.ShapeDtypeStruct((K * CH, CW), jnp.float32),
    mesh=_mesh,
    scratch_types=[
        pltpu.VMEM((16,), jnp.int32),
        pltpu.VMEM((16,), jnp.int32),
        pltpu.VMEM((K, CW), jnp.float32),
        pltpu.SemaphoreType.DMA,
    ],
)
def _subsample(in_hbm, idx_hbm, out_hbm, idx_v, rows_v, buf, sem):
    wid = lax.axis_index("s") * NC + lax.axis_index("c")
    pltpu.sync_copy(idx_hbm, idx_v)
    base = idx_v[...] * CH
    for i in range(CPW):
        c = wid * CPW + i
        rows_v[...] = base + c
        pltpu.async_copy(in_hbm.at[rows_v.at[pl.ds(0, K)]], buf, sem).wait()
        for j in range(K):
            pltpu.sync_copy(buf.at[j], out_hbm.at[j * CH + c])


def kernel(named_tensor, idx_to_keep):
    x = named_tensor.reshape(T * CH, CW)
    idx16 = jnp.pad(idx_to_keep.astype(jnp.int32), (0, 16 - K))
    out = _subsample(x, idx16)
    return out.reshape(K, 512, 512, 8)


# speedup vs baseline: 6.7712x; 6.7712x over previous
import functools

import jax
import jax.numpy as jnp
from jax import lax
from jax.experimental import pallas as pl
from jax.experimental.pallas import tpu as pltpu
from jax.experimental.pallas import tpu_sc as plsc

T = 16
K = 8
ROW = 512 * 512 * 8
NC, NS = 2, 16
NW = NC * NS
CH = 512
CW = ROW // CH
CPW = CH // NW

_mesh = plsc.VectorSubcoreMesh(core_axis_name="c", subcore_axis_name="s")


@functools.partial(
    pl.kernel,
    out_type=jax.ShapeDtypeStruct((K, CH, CW), jnp.float32),
    mesh=_mesh,
    scratch_types=[
        pltpu.VMEM((16,), jnp.int32),
        pltpu.VMEM((2, 16), jnp.int32),
        pltpu.VMEM((2, K, CW), jnp.float32),
        pltpu.SemaphoreType.DMA,
        pltpu.SemaphoreType.DMA,
        pltpu.SemaphoreType.DMA,
        pltpu.SemaphoreType.DMA,
    ],
)
def _subsample(in_hbm, idx_hbm, out_hbm, idx_v, rows_v, buf, g0, g1, s0, s1):
    wid = lax.axis_index("s") * NC + lax.axis_index("c")
    pltpu.sync_copy(idx_hbm, idx_v)
    base = idx_v[...] * CH
    gsem = (g0, g1)
    ssem = (s0, s1)

    def chunk(i):
        return wid * CPW + i

    def start_gather(i):
        b = i & 1
        rows_v[b, :] = base + chunk(i)
        return pltpu.async_copy(
            in_hbm.at[rows_v.at[b, pl.ds(0, K)]], buf.at[b], gsem[b]
        )

    def start_store(i):
        b = i & 1
        return pltpu.async_copy(
            buf.at[b], out_hbm.at[:, chunk(i)], ssem[b]
        )

    stores = [None, None]
    g = start_gather(0)
    for i in range(CPW):
        b = i & 1
        nb = (i + 1) & 1
        if i + 1 < CPW:
            if stores[nb] is not None:
                stores[nb].wait()
            g_next = start_gather(i + 1)
        g.wait()
        stores[b] = start_store(i)
        if i + 1 < CPW:
            g = g_next
    stores[0].wait()
    stores[1].wait()


def kernel(named_tensor, idx_to_keep):
    x = named_tensor.reshape(T * CH, CW)
    idx16 = jnp.pad(idx_to_keep.astype(jnp.int32), (0, 16 - K))
    out = _subsample(x, idx16)
    return out.reshape(K, 512, 512, 8)
